# baseline (speedup 1.0000x reference)
import functools

import jax
import jax.numpy as jnp
from jax import lax
from jax.experimental import pallas as pl
from jax.experimental.pallas import tpu as pltpu
from jax.experimental.pallas import tpu_sc as plsc

NC, NS = 2, 16
NW = NC * NS


def _sc_gather(buffer, idx16):
    B, L, D = buffer.shape
    spw = B // NW
    mesh = plsc.VectorSubcoreMesh(
        core_axis_name="c", subcore_axis_name="s",
        num_cores=NC, num_subcores=NS)

    @functools.partial(
        pl.kernel,
        out_type=jax.ShapeDtypeStruct((5, B, 128), jnp.float32),
        mesh=mesh,
        scratch_types=[
            pltpu.VMEM((spw, 16), jnp.int32),
            pltpu.VMEM((L, D), jnp.float32),
            pltpu.VMEM((L, D), jnp.float32),
            pltpu.VMEM((5, 32, 128), jnp.float32),
            pltpu.SemaphoreType.DMA,
            pltpu.SemaphoreType.DMA,
            pltpu.SemaphoreType.DMA,
        ],
        compiler_params=pltpu.CompilerParams(
            needs_layout_passes=False, use_tc_tiling_on_sc=False),
    )
    def gather_kernel(buf_hbm, idx_hbm, out_hbm, idx_v, sbuf0, sbuf1, xloc,
                      sem0, sem1, osem):
        wid = lax.axis_index("s") * NC + lax.axis_index("c")
        base = wid * spw
        pltpu.sync_copy(idx_hbm.at[pl.ds(base, spw)], idx_v)

        def fetch(s, sbuf, sem):
            return pltpu.async_copy(buf_hbm.at[base + s], sbuf, sem)

        lane = lax.iota(jnp.int32, 16)

        def select(s, sl, sbuf):
            vec = idx_v[s, pl.ds(0, 16)]
            for f in range(10):
                r = vec[f]
                p, q = f // 2, f % 2
                rows = jnp.broadcast_to(r, (16,))
                for c in (0, 16, 32, 44):
                    v = plsc.load_gather(sbuf, [rows, c + lane])
                    plsc.store_scatter(
                        xloc,
                        [jnp.broadcast_to(p, (16,)),
                         jnp.broadcast_to(sl, (16,)),
                         64 * q + c + lane],
                        v)

        def make_step(chunk):
            def step(g, _):
                s0 = chunk * 32 + g * 2
                ca = fetch(s0, sbuf0, sem0)
                cb = fetch(s0 + 1, sbuf1, sem1)
                ca.wait()
                select(s0, g * 2, sbuf0)
                cb.wait()
                select(s0 + 1, g * 2 + 1, sbuf1)
                return _
            return step

        for chunk in range(spw // 32):
            lax.fori_loop(0, 16, make_step(chunk), None)
            for p in range(5):
                pltpu.sync_copy(
                    xloc.at[p],
                    out_hbm.at[p, pl.ds(base + chunk * 32, 32)])

    return gather_kernel(buffer, idx16)


def _mlp(x5, w1p, b1, w2, b2, blk=512):
    _, B, _ = x5.shape
    h = w1p.shape[2]
    o = w2.shape[1]

    def body(x_ref, w1_ref, b1_ref, w2_ref, b2_ref, o_ref):
        acc = jnp.dot(x_ref[0], w1_ref[0], preferred_element_type=jnp.float32)
        for p in range(1, 5):
            acc = acc + jnp.dot(x_ref[p], w1_ref[p],
                                preferred_element_type=jnp.float32)
        hid = jnp.maximum(acc + b1_ref[...], 0.0)
        o_ref[...] = jnp.dot(hid, w2_ref[...],
                             preferred_element_type=jnp.float32) + b2_ref[...]

    return pl.pallas_call(
        body,
        grid=(B // blk,),
        in_specs=[
            pl.BlockSpec((5, blk, 128), lambda i: (0, i, 0)),
            pl.BlockSpec((5, 128, h), lambda i: (0, 0, 0)),
            pl.BlockSpec((1, h), lambda i: (0, 0)),
            pl.BlockSpec((h, o), lambda i: (0, 0)),
            pl.BlockSpec((1, o), lambda i: (0, 0)),
        ],
        out_specs=pl.BlockSpec((blk, o), lambda i: (i, 0)),
        out_shape=jax.ShapeDtypeStruct((B, o), jnp.float32),
    )(x5, w1p, b1.reshape(1, h), w2, b2.reshape(1, o))


def kernel(buffer, indexes, legal_actions, W1, b1, W2, b2):
    B, L, D = buffer.shape
    NF = indexes.shape[1]
    idx16 = jnp.pad(indexes.astype(jnp.int32), ((0, 0), (0, 16 - NF)))
    x5 = _sc_gather(buffer, idx16)
    w1p = jnp.pad(W1.reshape(NF, D, -1), ((0, 0), (0, 4), (0, 0)))
    w1p = w1p.reshape(NF // 2, 128, -1)
    out = _mlp(x5, w1p, b1, W2, b2)
    return out, legal_actions


# speedup vs baseline: 1.5519x; 1.5519x over previous
import functools

import jax
import jax.numpy as jnp
from jax import lax
from jax.experimental import pallas as pl
from jax.experimental.pallas import tpu as pltpu
from jax.experimental.pallas import tpu_sc as plsc

NC, NS = 2, 16
NW = NC * NS


def _sc_gather(buffer, idx16):
    B, L, D = buffer.shape
    spw = B // NW
    mesh = plsc.VectorSubcoreMesh(
        core_axis_name="c", subcore_axis_name="s",
        num_cores=NC, num_subcores=NS)

    @functools.partial(
        pl.kernel,
        out_type=jax.ShapeDtypeStruct((5, B, 128), jnp.float32),
        mesh=mesh,
        scratch_types=[
            pltpu.VMEM((spw, 16), jnp.int32),
            pltpu.VMEM((L, D), jnp.float32),
            pltpu.VMEM((L, D), jnp.float32),
            pltpu.VMEM((5, 32, 128), jnp.float32),
            pltpu.SemaphoreType.DMA,
            pltpu.SemaphoreType.DMA,
            pltpu.SemaphoreType.DMA,
        ],
        compiler_params=pltpu.CompilerParams(needs_layout_passes=False),
    )
    def gather_kernel(buf_hbm, idx_hbm, out_hbm, idx_v, sbuf0, sbuf1, xloc,
                      sem0, sem1, osem):
        wid = lax.axis_index("s") * NC + lax.axis_index("c")
        base = wid * spw
        pltpu.sync_copy(idx_hbm.at[pl.ds(base, spw)], idx_v)

        def fetch(s, sbuf, sem):
            return pltpu.async_copy(buf_hbm.at[base + s], sbuf, sem)

        lane = lax.iota(jnp.int32, 16)

        def select(s, sl, sbuf):
            vec = idx_v[s, pl.ds(0, 16)]
            for f in range(10):
                r = vec[f]
                p, q = f // 2, f % 2
                rows = jnp.broadcast_to(r, (16,))
                for c in (0, 16, 32, 44):
                    v = plsc.load_gather(sbuf, [rows, c + lane])
                    plsc.store_scatter(
                        xloc,
                        [jnp.broadcast_to(p, (16,)),
                         jnp.broadcast_to(sl, (16,)),
                         64 * q + c + lane],
                        v)

        def make_step(chunk):
            def step(g, _):
                s0 = chunk * 32 + g * 2
                ca = fetch(s0, sbuf0, sem0)
                cb = fetch(s0 + 1, sbuf1, sem1)
                ca.wait()
                select(s0, g * 2, sbuf0)
                cb.wait()
                select(s0 + 1, g * 2 + 1, sbuf1)
                return _
            return step

        for chunk in range(spw // 32):
            lax.fori_loop(0, 16, make_step(chunk), None)
            for p in range(5):
                pltpu.sync_copy(
                    xloc.at[p],
                    out_hbm.at[p, pl.ds(base + chunk * 32, 32)])

    return gather_kernel(buffer, idx16)


def _mlp(x5, w1p, b1, w2, b2, blk=512):
    _, B, _ = x5.shape
    h = w1p.shape[2]
    o = w2.shape[1]

    def body(x_ref, w1_ref, b1_ref, w2_ref, b2_ref, o_ref):
        acc = jnp.dot(x_ref[0], w1_ref[0], preferred_element_type=jnp.float32)
        for p in range(1, 5):
            acc = acc + jnp.dot(x_ref[p], w1_ref[p],
                                preferred_element_type=jnp.float32)
        hid = jnp.maximum(acc + b1_ref[...], 0.0)
        o_ref[...] = jnp.dot(hid, w2_ref[...],
                             preferred_element_type=jnp.float32) + b2_ref[...]

    return pl.pallas_call(
        body,
        grid=(B // blk,),
        in_specs=[
            pl.BlockSpec((5, blk, 128), lambda i: (0, i, 0)),
            pl.BlockSpec((5, 128, h), lambda i: (0, 0, 0)),
            pl.BlockSpec((1, h), lambda i: (0, 0)),
            pl.BlockSpec((h, o), lambda i: (0, 0)),
            pl.BlockSpec((1, o), lambda i: (0, 0)),
        ],
        out_specs=pl.BlockSpec((blk, o), lambda i: (i, 0)),
        out_shape=jax.ShapeDtypeStruct((B, o), jnp.float32),
    )(x5, w1p, b1.reshape(1, h), w2, b2.reshape(1, o))


def kernel(buffer, indexes, legal_actions, W1, b1, W2, b2):
    B, L, D = buffer.shape
    NF = indexes.shape[1]
    idx16 = jnp.pad(indexes.astype(jnp.int32), ((0, 0), (0, 16 - NF)))
    w1p = jnp.pad(W1.reshape(NF, D, -1), ((0, 0), (0, 4), (0, 0)))
    w1p = w1p.reshape(NF // 2, 128, -1)
    h = B // 2
    outs = []
    for lo in (0, h):
        x5 = _sc_gather(buffer[lo:lo + h], idx16[lo:lo + h])
        outs.append(_mlp(x5, w1p, b1, W2, b2))
    out = jnp.concatenate(outs, axis=0)
    return out, legal_actions


# speedup vs baseline: 1.7286x; 1.1139x over previous
import functools

import jax
import jax.numpy as jnp
from jax import lax
from jax.experimental import pallas as pl
from jax.experimental.pallas import tpu as pltpu
from jax.experimental.pallas import tpu_sc as plsc

NC, NS = 2, 16
NW = NC * NS


def _sc_gather(buffer, idx16):
    B, L, D = buffer.shape
    spw = B // NW
    mesh = plsc.VectorSubcoreMesh(
        core_axis_name="c", subcore_axis_name="s",
        num_cores=NC, num_subcores=NS)

    @functools.partial(
        pl.kernel,
        out_type=jax.ShapeDtypeStruct((5, B, 128), jnp.float32),
        mesh=mesh,
        scratch_types=[
            pltpu.VMEM((spw, 16), jnp.int32),
            pltpu.VMEM((L, D), jnp.float32),
            pltpu.VMEM((L, D), jnp.float32),
            pltpu.VMEM((5, 32, 128), jnp.float32),
            pltpu.SemaphoreType.DMA,
            pltpu.SemaphoreType.DMA,
            pltpu.SemaphoreType.DMA,
        ],
        compiler_params=pltpu.CompilerParams(needs_layout_passes=False),
    )
    def gather_kernel(buf_hbm, idx_hbm, out_hbm, idx_v, sbuf0, sbuf1, xloc,
                      sem0, sem1, osem):
        wid = lax.axis_index("s") * NC + lax.axis_index("c")
        base = wid * spw
        pltpu.sync_copy(idx_hbm.at[pl.ds(base, spw)], idx_v)

        def fetch(s, sbuf, sem):
            return pltpu.async_copy(buf_hbm.at[base + s], sbuf, sem)

        lane = lax.iota(jnp.int32, 16)

        def select(s, sl, sbuf):
            vec = idx_v[s, pl.ds(0, 16)]
            for f in range(10):
                r = vec[f]
                p, q = f // 2, f % 2
                rows = jnp.broadcast_to(r, (16,))
                for c in (0, 16, 32, 44):
                    v = plsc.load_gather(sbuf, [rows, c + lane])
                    plsc.store_scatter(
                        xloc,
                        [jnp.broadcast_to(p, (16,)),
                         jnp.broadcast_to(sl, (16,)),
                         64 * q + c + lane],
                        v)

        def make_step(chunk):
            def step(g, _):
                s0 = chunk * 32 + g * 2
                ca = fetch(s0, sbuf0, sem0)
                cb = fetch(s0 + 1, sbuf1, sem1)
                ca.wait()
                select(s0, g * 2, sbuf0)
                cb.wait()
                select(s0 + 1, g * 2 + 1, sbuf1)
                return _
            return step

        for chunk in range(spw // 32):
            lax.fori_loop(0, 16, make_step(chunk), None)
            for p in range(5):
                pltpu.sync_copy(
                    xloc.at[p],
                    out_hbm.at[p, pl.ds(base + chunk * 32, 32)])

    return gather_kernel(buffer, idx16)


def _mlp(x5, w1p, b1, w2, b2, blk=512):
    _, B, _ = x5.shape
    h = w1p.shape[2]
    o = w2.shape[1]

    def body(x_ref, w1_ref, b1_ref, w2_ref, b2_ref, o_ref):
        acc = jnp.dot(x_ref[0], w1_ref[0], preferred_element_type=jnp.float32)
        for p in range(1, 5):
            acc = acc + jnp.dot(x_ref[p], w1_ref[p],
                                preferred_element_type=jnp.float32)
        hid = jnp.maximum(acc + b1_ref[...], 0.0)
        o_ref[...] = jnp.dot(hid, w2_ref[...],
                             preferred_element_type=jnp.float32) + b2_ref[...]

    return pl.pallas_call(
        body,
        grid=(B // blk,),
        in_specs=[
            pl.BlockSpec((5, blk, 128), lambda i: (0, i, 0)),
            pl.BlockSpec((5, 128, h), lambda i: (0, 0, 0)),
            pl.BlockSpec((1, h), lambda i: (0, 0)),
            pl.BlockSpec((h, o), lambda i: (0, 0)),
            pl.BlockSpec((1, o), lambda i: (0, 0)),
        ],
        out_specs=pl.BlockSpec((blk, o), lambda i: (i, 0)),
        out_shape=jax.ShapeDtypeStruct((B, o), jnp.float32),
    )(x5, w1p, b1.reshape(1, h), w2, b2.reshape(1, o))


def kernel(buffer, indexes, legal_actions, W1, b1, W2, b2):
    B, L, D = buffer.shape
    NF = indexes.shape[1]
    idx16 = jnp.pad(indexes.astype(jnp.int32), ((0, 0), (0, 16 - NF)))
    x5 = _sc_gather(buffer, idx16)
    w1p = jnp.pad(W1.reshape(NF, D, -1), ((0, 0), (0, 4), (0, 0)))
    w1p = w1p.reshape(NF // 2, 128, -1)
    out = _mlp(x5, w1p, b1, W2, b2)
    return out, legal_actions
